# baseline (speedup 1.0000x reference)
import functools

import jax
import jax.numpy as jnp
from jax import lax
from jax.experimental import pallas as pl
from jax.experimental.pallas import tpu as pltpu
from jax.experimental.pallas import tpu_sc as plsc

_B = 16384
_D_USER, _D_COUNTRY, _D_DEVICE, _D_DENSE = 32, 16, 16, 128
_TOTAL = _D_USER + _D_COUNTRY + _D_DEVICE + _D_DENSE
_OUT_D = 128
_EPS = 1.1920928955078125e-07

_NC, _NS = 2, 16
_NW = _NC * _NS
_BPW = _B // _NW


def _sc_gather_body(uid_hbm, cid_hbm, did_hbm, eu_hbm, ec_hbm, ed_hbm,
                    out_u, out_c, out_d,
                    uidx_v, cidx_v, didx_v, urows_v, crows_v, drows_v,
                    sem_u, sem_c, sem_d):
    wid = lax.axis_index("s") * _NC + lax.axis_index("c")
    base = wid * _BPW
    pltpu.sync_copy(uid_hbm.at[pl.ds(base, _BPW)], uidx_v)
    pltpu.sync_copy(cid_hbm.at[pl.ds(base, _BPW)], cidx_v)
    pltpu.sync_copy(did_hbm.at[pl.ds(base, _BPW)], didx_v)
    cp_u = pltpu.async_copy(eu_hbm.at[uidx_v], urows_v, sem_u)
    cp_c = pltpu.async_copy(ec_hbm.at[cidx_v], crows_v, sem_c)
    cp_d = pltpu.async_copy(ed_hbm.at[didx_v], drows_v, sem_d)
    cp_u.wait()
    cp_c.wait()
    cp_d.wait()
    pltpu.sync_copy(urows_v, out_u.at[pl.ds(base, _BPW)])
    pltpu.sync_copy(crows_v, out_c.at[pl.ds(base, _BPW)])
    pltpu.sync_copy(drows_v, out_d.at[pl.ds(base, _BPW)])


def _sc_gather(user_id, country, device, emb_user, emb_country, emb_device):
    return pl.kernel(
        _sc_gather_body,
        out_type=[
            jax.ShapeDtypeStruct((_B, _D_USER), jnp.float32),
            jax.ShapeDtypeStruct((_B, _D_COUNTRY), jnp.float32),
            jax.ShapeDtypeStruct((_B, _D_DEVICE), jnp.float32),
        ],
        mesh=plsc.VectorSubcoreMesh(core_axis_name="c", subcore_axis_name="s"),
        compiler_params=pltpu.CompilerParams(use_tc_tiling_on_sc=False),
        scratch_types=[
            pltpu.VMEM((_BPW,), jnp.int32),
            pltpu.VMEM((_BPW,), jnp.int32),
            pltpu.VMEM((_BPW,), jnp.int32),
            pltpu.VMEM((_BPW, _D_USER), jnp.float32),
            pltpu.VMEM((_BPW, _D_COUNTRY), jnp.float32),
            pltpu.VMEM((_BPW, _D_DEVICE), jnp.float32),
            pltpu.SemaphoreType.DMA,
            pltpu.SemaphoreType.DMA,
            pltpu.SemaphoreType.DMA,
        ],
    )(user_id, country, device, emb_user, emb_country, emb_device)


def _tc_body(u_ref, c_ref, d_ref, x_ref, rw_ref, w_ref, b_ref, out_ref):
    u = u_ref[...]
    c = c_ref[...]
    d = d_ref[...]
    x = x_ref[...]
    ssq = (jnp.sum(u * u, axis=1, keepdims=True)
           + jnp.sum(c * c, axis=1, keepdims=True)
           + jnp.sum(d * d, axis=1, keepdims=True)
           + jnp.sum(x * x, axis=1, keepdims=True))
    scale = lax.rsqrt(ssq * (1.0 / _TOTAL) + _EPS)
    ws = w_ref[...] * rw_ref[...]
    s0, s1, s2 = _D_USER, _D_USER + _D_COUNTRY, _D_USER + _D_COUNTRY + _D_DEVICE
    acc = jnp.dot(u, ws[0:s0], preferred_element_type=jnp.float32)
    acc += jnp.dot(c, ws[s0:s1], preferred_element_type=jnp.float32)
    acc += jnp.dot(d, ws[s1:s2], preferred_element_type=jnp.float32)
    acc += jnp.dot(x, ws[s2:_TOTAL], preferred_element_type=jnp.float32)
    out_ref[...] = acc * scale + b_ref[...]


def _tc_norm_linear(e_user, e_country, e_device, dense_profile, rms_weight, W, b,
                    blk=1024):
    grid = _B // blk
    rw = rms_weight.reshape(_TOTAL, 1)
    b2 = b.reshape(1, _OUT_D)
    return pl.pallas_call(
        _tc_body,
        grid=(grid,),
        in_specs=[
            pl.BlockSpec((blk, _D_USER), lambda i: (i, 0)),
            pl.BlockSpec((blk, _D_COUNTRY), lambda i: (i, 0)),
            pl.BlockSpec((blk, _D_DEVICE), lambda i: (i, 0)),
            pl.BlockSpec((blk, _D_DENSE), lambda i: (i, 0)),
            pl.BlockSpec((_TOTAL, 1), lambda i: (0, 0)),
            pl.BlockSpec((_TOTAL, _OUT_D), lambda i: (0, 0)),
            pl.BlockSpec((1, _OUT_D), lambda i: (0, 0)),
        ],
        out_specs=pl.BlockSpec((blk, _OUT_D), lambda i: (i, 0)),
        out_shape=jax.ShapeDtypeStruct((_B, _OUT_D), jnp.float32),
    )(e_user, e_country, e_device, dense_profile, rw, W, b2)


def kernel(user_id, country, device, dense_profile, emb_user, emb_country,
           emb_device, rms_weight, W, b):
    e_user, e_country, e_device = _sc_gather(
        user_id.astype(jnp.int32), country, device,
        emb_user, emb_country, emb_device)
    return _tc_norm_linear(e_user, e_country, e_device, dense_profile,
                           rms_weight, W, b)


# speedup vs baseline: 3.5393x; 3.5393x over previous
import functools

import jax
import jax.numpy as jnp
from jax import lax
from jax.experimental import pallas as pl
from jax.experimental.pallas import tpu as pltpu
from jax.experimental.pallas import tpu_sc as plsc

_B = 16384
_D_USER, _D_COUNTRY, _D_DEVICE, _D_DENSE = 32, 16, 16, 128
_TOTAL = _D_USER + _D_COUNTRY + _D_DEVICE + _D_DENSE
_OUT_D = 128
_EPS = 1.1920928955078125e-07

_NC, _NS = 2, 16
_NW = _NC * _NS
_BPW = _B // _NW


_V_USER = 1000000
_BPS = _B // _NS
_STAGE = 65536
_STAGE_TAIL = _V_USER - 15 * _STAGE


def _sc_user_body(uid_hbm, eu_t_hbm, out_flat, uidx_v, dst_v, row_sh, sem):
    c = lax.axis_index("c")
    s = lax.axis_index("s")
    pltpu.sync_copy(uid_hbm.at[pl.ds(s * _BPS, _BPS)], uidx_v)

    def step(j_local, carry):
        j = c * _NS + j_local
        @pl.when(s < 15)
        def _():
            pltpu.sync_copy(
                eu_t_hbm.at[pl.ds(j, 1), pl.ds(s * _STAGE, _STAGE)],
                row_sh.at[pl.ds(0, 1), pl.ds(s * _STAGE, _STAGE)])

        @pl.when(s == 15)
        def _():
            pltpu.sync_copy(
                eu_t_hbm.at[pl.ds(j, 1), pl.ds(15 * _STAGE, _STAGE_TAIL)],
                row_sh.at[pl.ds(0, 1), pl.ds(15 * _STAGE, _STAGE_TAIL)])

        plsc.subcore_barrier()
        pltpu.async_copy(row_sh.at[0].at[uidx_v], dst_v, sem).wait()
        pltpu.sync_copy(dst_v, out_flat.at[pl.ds(j * _B + s * _BPS, _BPS)])
        plsc.subcore_barrier()
        return carry

    lax.fori_loop(0, _NS, step, 0)


def _sc_user_gather(user_id, emb_user_t):
    out = pl.kernel(
        _sc_user_body,
        out_type=jax.ShapeDtypeStruct((_D_USER * _B,), jnp.float32),
        mesh=plsc.VectorSubcoreMesh(core_axis_name="c", subcore_axis_name="s"),
        compiler_params=pltpu.CompilerParams(use_tc_tiling_on_sc=True),
        scratch_types=[
            pltpu.VMEM((_BPS,), jnp.int32),
            pltpu.VMEM((_BPS,), jnp.float32),
            pltpu.VMEM_SHARED((1, _V_USER), jnp.float32),
            pltpu.SemaphoreType.DMA,
        ],
    )(user_id, emb_user_t)
    return out.reshape(_D_USER, _B)


def _sc_small_body(cid_hbm, did_hbm, ec_hbm, ed_hbm, out_c, out_d,
                   cidx_v, didx_v, crows_v, drows_v, sem_c, sem_d):
    wid = lax.axis_index("s") * _NC + lax.axis_index("c")
    base = wid * _BPW
    pltpu.sync_copy(cid_hbm.at[pl.ds(base, _BPW)], cidx_v)
    pltpu.sync_copy(did_hbm.at[pl.ds(base, _BPW)], didx_v)
    cp_c = pltpu.async_copy(ec_hbm.at[cidx_v], crows_v, sem_c)
    cp_d = pltpu.async_copy(ed_hbm.at[didx_v], drows_v, sem_d)
    cp_c.wait()
    cp_d.wait()
    pltpu.sync_copy(crows_v, out_c.at[pl.ds(base, _BPW)])
    pltpu.sync_copy(drows_v, out_d.at[pl.ds(base, _BPW)])


def _sc_small_gather(country, device, emb_country, emb_device):
    return pl.kernel(
        _sc_small_body,
        out_type=[
            jax.ShapeDtypeStruct((_B, _D_COUNTRY), jnp.float32),
            jax.ShapeDtypeStruct((_B, _D_DEVICE), jnp.float32),
        ],
        mesh=plsc.VectorSubcoreMesh(core_axis_name="c", subcore_axis_name="s"),
        compiler_params=pltpu.CompilerParams(use_tc_tiling_on_sc=False),
        scratch_types=[
            pltpu.VMEM((_BPW,), jnp.int32),
            pltpu.VMEM((_BPW,), jnp.int32),
            pltpu.VMEM((_BPW, _D_COUNTRY), jnp.float32),
            pltpu.VMEM((_BPW, _D_DEVICE), jnp.float32),
            pltpu.SemaphoreType.DMA,
            pltpu.SemaphoreType.DMA,
        ],
    )(country, device, emb_country, emb_device)


def _tc_body(ut_ref, c_ref, d_ref, x_ref, rw_ref, w_ref, b_ref, out_ref):
    ut = ut_ref[...]
    c = c_ref[...]
    d = d_ref[...]
    x = x_ref[...]
    ones = jnp.ones((_D_USER, 1), jnp.float32)
    ssq_u = lax.dot_general(ut * ut, ones, (((0,), (0,)), ((), ())),
                            preferred_element_type=jnp.float32)
    ssq = (ssq_u
           + jnp.sum(c * c, axis=1, keepdims=True)
           + jnp.sum(d * d, axis=1, keepdims=True)
           + jnp.sum(x * x, axis=1, keepdims=True))
    scale = lax.rsqrt(ssq * (1.0 / _TOTAL) + _EPS)
    ws = w_ref[...] * rw_ref[...]
    s0, s1, s2 = _D_USER, _D_USER + _D_COUNTRY, _D_USER + _D_COUNTRY + _D_DEVICE
    acc = lax.dot_general(ut, ws[0:s0], (((0,), (0,)), ((), ())),
                          preferred_element_type=jnp.float32)
    acc += jnp.dot(c, ws[s0:s1], preferred_element_type=jnp.float32)
    acc += jnp.dot(d, ws[s1:s2], preferred_element_type=jnp.float32)
    acc += jnp.dot(x, ws[s2:_TOTAL], preferred_element_type=jnp.float32)
    out_ref[...] = acc * scale + b_ref[...]


def _tc_norm_linear(e_user_t, e_country, e_device, dense_profile, rms_weight,
                    W, b, blk=1024):
    grid = _B // blk
    rw = rms_weight.reshape(_TOTAL, 1)
    b2 = b.reshape(1, _OUT_D)
    return pl.pallas_call(
        _tc_body,
        grid=(grid,),
        in_specs=[
            pl.BlockSpec((_D_USER, blk), lambda i: (0, i)),
            pl.BlockSpec((blk, _D_COUNTRY), lambda i: (i, 0)),
            pl.BlockSpec((blk, _D_DEVICE), lambda i: (i, 0)),
            pl.BlockSpec((blk, _D_DENSE), lambda i: (i, 0)),
            pl.BlockSpec((_TOTAL, 1), lambda i: (0, 0)),
            pl.BlockSpec((_TOTAL, _OUT_D), lambda i: (0, 0)),
            pl.BlockSpec((1, _OUT_D), lambda i: (0, 0)),
        ],
        out_specs=pl.BlockSpec((blk, _OUT_D), lambda i: (i, 0)),
        out_shape=jax.ShapeDtypeStruct((_B, _OUT_D), jnp.float32),
    )(e_user_t, e_country, e_device, dense_profile, rw, W, b2)


def kernel(user_id, country, device, dense_profile, emb_user, emb_country,
           emb_device, rms_weight, W, b):
    e_user_t = _sc_user_gather(user_id.astype(jnp.int32), emb_user.T)
    e_country, e_device = _sc_small_gather(country, device, emb_country,
                                           emb_device)
    return _tc_norm_linear(e_user_t, e_country, e_device, dense_profile,
                           rms_weight, W, b)
